# baseline (speedup 1.0000x reference)
import functools

import numpy as np

import jax
import jax.numpy as jnp
from jax import lax
from jax.experimental import pallas as pl
from jax.experimental.pallas import tpu as pltpu
from jax.experimental.pallas import tpu_sc as plsc

N = 8192
CIN = 6
D = 256
H = 8
DH = D // H
K = 16
M = N // 4
HID = 64
SCALE = DH ** -0.5

bf16 = jnp.bfloat16
f32 = jnp.float32
i32 = jnp.int32


def _mm(a, b, prec=None):
    if prec is None:
        a = a.astype(bf16)
        b = b.astype(bf16)
    return lax.dot_general(a, b, (((a.ndim - 1,), (0,)), ((), ())),
                           preferred_element_type=f32,
                           precision=prec)


def _ln(x, g, b):
    mu = jnp.mean(x, -1, keepdims=True)
    v = jnp.mean((x - mu) ** 2, -1, keepdims=True)
    return g * (x - mu) / jnp.sqrt(v + 1e-5) + b


def _full(shape):
    nd = len(shape)
    return pl.BlockSpec(shape, lambda i: (0,) * nd)


def _rows(bshape):
    nd = len(bshape)
    return pl.BlockSpec(bshape, lambda i: (i,) + (0,) * (nd - 1))


def _embed_body(pos_ref, feat_ref,
                cw1, cb1, cg, cbe, cw2, cb2,
                fw1, fb1, fg, fbe, fw2, fb2,
                fuwa, fuwb, fub, fug, fube,
                qw, qb, kw, kb, vw, vb,
                x_ref, q_ref, k_ref, v_ref):
    pe = _mm(jax.nn.gelu(_ln(_mm(pos_ref[...], cw1[...]) + cb1[...],
                             cg[...], cbe[...])), cw2[...]) + cb2[...]
    fe = _mm(jax.nn.gelu(_ln(_mm(feat_ref[...], fw1[...]) + fb1[...],
                             fg[...], fbe[...])), fw2[...]) + fb2[...]
    fu = _mm(pe, fuwa[...]) + _mm(fe, fuwb[...]) + fub[...]
    x = jax.nn.gelu(_ln(fu, fug[...], fube[...]))
    x_ref[...] = x
    q_ref[...] = _mm(x, qw[...]) + qb[...]
    k_ref[...] = _mm(x, kw[...]) + kb[...]
    v_ref[...] = _mm(x, vw[...]) + vb[...]


def _embed(pos, feat, w):
    R = 512
    outs = [jax.ShapeDtypeStruct((N, D), f32)] * 4
    in_arrs = [pos, feat] + w
    in_specs = [_rows((R, 3)), _rows((R, CIN))] + [_full(a.shape) for a in w]
    return pl.pallas_call(
        _embed_body,
        grid=(N // R,),
        in_specs=in_specs,
        out_specs=[_rows((R, D))] * 4,
        out_shape=outs,
    )(*in_arrs)


_RK = 64
_CH = 1024
_NCH = N // _CH
_NL = 8
_INF = np.int32(0x7FFFFFFF)
_BIGP = np.int32(1 << 30)


def _knn_body(pos_ref, post_ref, sqr_ref, sqc_ref, out_ref):
    pos_b = pos_ref[...].astype(bf16)
    sqr = sqr_ref[...]
    lane = lax.broadcasted_iota(i32, (_RK, _CH), 1)
    g_local = lane >> 7

    anchor = (lane & 3) == 0

    def dist_chunk(c):
        off = pl.multiple_of(c * _CH, _CH)
        ptc = post_ref[:, pl.ds(off, _CH)].astype(bf16)
        return sqr + sqc_ref[:, pl.ds(off, _CH)] - 2.0 * lax.dot_general(
            pos_b, ptc, (((1,), (0,)), ((), ())), preferred_element_type=f32)

    def chunk(c, carry):
        lists, amin = carry
        lists = list(lists)
        d = dist_chunk(c)
        amin = jnp.minimum(amin, jnp.min(jnp.where(anchor, d, jnp.inf),
                                         axis=1, keepdims=True))
        b = lax.bitcast_convert_type(d + 0.5, i32)
        keys = lax.bitcast_convert_type(
            (b & jnp.int32(-64)) | (g_local + c * (_CH // 128)), f32)
        for s in range(_CH // 128):
            kg = keys[:, s * 128:(s + 1) * 128]
            for j in range(_NL):
                lo = jnp.minimum(lists[j], kg)
                kg = jnp.maximum(lists[j], kg)
                lists[j] = lo
        return tuple(lists), amin

    init = (tuple(jnp.full((_RK, 128), jnp.inf, f32) for _ in range(_NL)),
            jnp.full((_RK, 1), jnp.inf, f32))
    lists, amin = lax.fori_loop(0, _NCH, chunk, init)

    def upchunk(c, acc):
        d = dist_chunk(c)
        colc = jnp.where(anchor & (d == amin), lane + c * _CH, _BIGP)
        return jnp.minimum(acc, jnp.min(colc, axis=1, keepdims=True))

    up_col = lax.fori_loop(0, _NCH, upchunk,
                           jnp.full((_RK, 1), _BIGP, i32)) >> 2

    cand = jnp.concatenate(lists, axis=1)
    lane_c = lax.broadcasted_iota(i32, (_RK, _NL * 128), 1)
    acc = jnp.zeros((_RK, 24), i32)
    kio = lax.broadcasted_iota(i32, (_RK, 24), 1)
    for kk in range(K):
        m = jnp.min(cand, axis=1, keepdims=True)
        p = jnp.min(jnp.where(cand == m, lane_c, _BIGP), axis=1, keepdims=True)
        col = (lax.bitcast_convert_type(m, i32) & 63) * 128 + (p & 127)
        acc = jnp.where(kio == kk, col, acc)
        cand = jnp.where(lane_c == p, jnp.inf, cand)
    acc = jnp.where(kio == K, up_col, acc)
    out_ref[...] = acc


def _knn(pos8, post8, sqr, sqc):
    return pl.pallas_call(
        _knn_body,
        grid=(N // _RK,),
        in_specs=[_rows((_RK, 8)), _full((8, N)), _rows((_RK, 1)),
                  _full((1, N))],
        out_specs=_rows((_RK, 24)),
        out_shape=jax.ShapeDtypeStruct((N, 24), i32),
    )(pos8, post8, sqr, sqc)


_NW = 32


def _gather_multi(tables, idx):
    B = idx.shape[0]
    T = len(tables)
    bw = B // _NW
    wsum = sum(t.shape[1] for t in tables)
    CH = 128 if wsum <= 384 else 64
    nch = bw // CH
    idx2d = idx.reshape(B // CH, CH)
    mesh = plsc.VectorSubcoreMesh(core_axis_name="c", subcore_axis_name="s")

    scratch = [pltpu.VMEM((nch, CH), i32)]
    for t in tables:
        scratch += [pltpu.VMEM((CH, t.shape[1]), t.dtype)] * 2
    scratch += [pltpu.SemaphoreType.DMA] * (4 * T)

    @functools.partial(
        pl.kernel,
        out_type=[jax.ShapeDtypeStruct((B, t.shape[1]), t.dtype)
                  for t in tables],
        mesh=mesh,
        scratch_types=scratch,
    )
    def gk(*refs):
        tabs = refs[:T]
        idx_hbm = refs[T]
        outs = refs[T + 1:2 * T + 1]
        idx_v = refs[2 * T + 1]
        bufs = refs[2 * T + 2:2 * T + 2 + 2 * T]
        sems = refs[2 * T + 2 + 2 * T:]
        rows = [(bufs[2 * i], bufs[2 * i + 1]) for i in range(T)]
        sg = [(sems[4 * i], sems[4 * i + 1]) for i in range(T)]
        ss = [(sems[4 * i + 2], sems[4 * i + 3]) for i in range(T)]
        wid = lax.axis_index("s") * 2 + lax.axis_index("c")
        base = wid * bw
        pltpu.sync_copy(idx_hbm.at[pl.ds(wid * nch, nch)], idx_v)
        for t in range(T):
            pltpu.async_copy(tabs[t].at[idx_v.at[0]], rows[t][0], sg[t][0])

        def body(c2, carry):
            c = 2 * c2
            for t in range(T):
                pltpu.make_async_copy(tabs[t].at[idx_v.at[c]], rows[t][0],
                                      sg[t][0]).wait()

            @pl.when(c2 > 0)
            def _():
                for t in range(T):
                    pltpu.make_async_copy(rows[t][1],
                                          outs[t].at[pl.ds(base, CH)],
                                          ss[t][1]).wait()

            for t in range(T):
                pltpu.async_copy(tabs[t].at[idx_v.at[c + 1]], rows[t][1],
                                 sg[t][1])
            for t in range(T):
                pltpu.async_copy(rows[t][0],
                                 outs[t].at[pl.ds(base + c * CH, CH)],
                                 ss[t][0])
            for t in range(T):
                pltpu.make_async_copy(tabs[t].at[idx_v.at[c + 1]], rows[t][1],
                                      sg[t][1]).wait()

            @pl.when(c2 < nch // 2 - 1)
            def _():
                for t in range(T):
                    pltpu.make_async_copy(rows[t][0],
                                          outs[t].at[pl.ds(base, CH)],
                                          ss[t][0]).wait()
                    pltpu.async_copy(tabs[t].at[idx_v.at[c + 2]], rows[t][0],
                                     sg[t][0])

            for t in range(T):
                pltpu.async_copy(rows[t][1],
                                 outs[t].at[pl.ds(base + (c + 1) * CH, CH)],
                                 ss[t][1])
            return carry

        lax.fori_loop(0, nch // 2, body, 0)
        for t in range(T):
            pltpu.make_async_copy(rows[t][0], outs[t].at[pl.ds(base, CH)],
                                  ss[t][0]).wait()
            pltpu.make_async_copy(rows[t][1], outs[t].at[pl.ds(base, CH)],
                                  ss[t][1]).wait()

    out = gk(*tables, idx2d)
    return out if isinstance(out, (list, tuple)) else [out]


def _gather_rows(table, idx):
    return _gather_multi([table], idx)[0]


_RA = 128


def _attn_body(q_ref, kg_ref, vg_ref, pg_ref, posp_ref, x_ref,
               rw1, rb1, rw2, rb2, ow, ob, lag, labe, s_ref, st_ref,
               x2_ref):
    RK = _RA * K
    pos_rep = jnp.broadcast_to(posp_ref[...][:, None, :],
                               (_RA, K, 128)).reshape(RK, 128)
    rel = pg_ref[...] - pos_rep
    bias = _mm(jax.nn.gelu(_mm(rel, rw1[...]) + rb1[...]), rw2[...]) + rb2[...]

    q_rep = jnp.broadcast_to(q_ref[...][:, None, :],
                             (_RA, K, D)).reshape(RK, D)
    qb = q_rep.astype(bf16).astype(f32)
    kb = kg_ref[...].astype(bf16).astype(f32)
    prod = qb * kb
    logits = _mm(prod, s_ref[...], prec=lax.Precision.HIGHEST) * SCALE + bias
    l3 = logits.reshape(_RA, K, H)
    mx = jnp.max(l3, axis=1, keepdims=True)
    e = jnp.exp(l3 - mx)
    sm = (e / jnp.sum(e, axis=1, keepdims=True)).reshape(RK, H)
    a_exp = _mm(sm.astype(bf16).astype(f32), st_ref[...],
                prec=lax.Precision.HIGHEST)
    vb = vg_ref[...].astype(bf16).astype(f32)
    o = jnp.sum((a_exp * vb).reshape(_RA, K, D), axis=1)
    out = _mm(o, ow[...]) + ob[...]
    x2_ref[...] = _ln(x_ref[...] + out, lag[...], labe[...])


def _local_attn(qp, kg, vg, pg, posp16, x, w, off):
    smat = jnp.repeat(jnp.eye(H, dtype=f32), DH, axis=0)
    stmat = smat.T
    nrows = kg.shape[0] // K
    ob = off // _RA
    offrows = lambda bshape: pl.BlockSpec(
        bshape, lambda i: (i + ob,) + (0,) * (len(bshape) - 1))
    in_arrs = [qp, kg, vg, pg, posp16, x] + w + [smat, stmat]
    in_specs = ([offrows((_RA, D)), _rows((_RA * K, D)), _rows((_RA * K, D)),
                 _rows((_RA * K, 128)), offrows((_RA, 128)),
                 offrows((_RA, D))] +
                [_full(a.shape) for a in w] +
                [_full((D, H)), _full((H, D))])
    return pl.pallas_call(
        _attn_body,
        grid=(nrows // _RA,),
        in_specs=in_specs,
        out_specs=_rows((_RA, D)),
        out_shape=jax.ShapeDtypeStruct((nrows, D), f32),
    )(*in_arrs)


def _sa_body(xg_ref, pga_ref, posa_ref, swx, swp, sb, sg, sbe, xd_ref):
    RK = _RA * K
    pos_rep = jnp.broadcast_to(posa_ref[...][:, None, :],
                               (_RA, K, 128)).reshape(RK, 128)
    rel = pga_ref[...] - pos_rep
    gin = _mm(xg_ref[...], swx[...]) + _mm(rel, swp[...]) + sb[...]
    g = jax.nn.gelu(_ln(gin, sg[...], sbe[...]))
    xd_ref[...] = jnp.max(g.reshape(_RA, K, D), axis=1)


def _set_abs(xg, pga, posa16, w):
    in_arrs = [xg, pga, posa16] + w
    in_specs = ([_rows((_RA * K, D)), _rows((_RA * K, 128)),
                 _rows((_RA, 128))] + [_full(a.shape) for a in w])
    return pl.pallas_call(
        _sa_body,
        grid=(M // _RA,),
        in_specs=in_specs,
        out_specs=_rows((_RA, D)),
        out_shape=jax.ShapeDtypeStruct((M, D), f32),
    )(*in_arrs)


def _ga_attn_body(xd_ref, qw, qb, kw, kb, vw, vb, og_ref):
    qh = (_mm(xd_ref[...], qw[0]) + qb[0]).astype(bf16)
    kh = (_mm(xd_ref[...], kw[0]) + kb[0]).astype(bf16)
    vh = _mm(xd_ref[...], vw[0]) + vb[0]
    s = lax.dot_general(qh, kh, (((1,), (1,)), ((), ())),
                        preferred_element_type=f32) * SCALE
    mx = jnp.max(s, axis=1, keepdims=True)
    e = jnp.exp(s - mx)
    a = e / jnp.sum(e, axis=1, keepdims=True)
    og_ref[0] = _mm(a, vh)


def _ga_post_body(xd_ref, og_ref, gow, gob, n1g, n1be, f1w, f1b, f2w, f2b,
                  n2g, n2be, xd2_ref):
    og = gob[...]
    for h in range(H):
        og = og + _mm(og_ref[h], gow[h])
    xd1 = _ln(xd_ref[...] + og, n1g[...], n1be[...])
    ff = _mm(jax.nn.gelu(_mm(xd1, f1w[...]) + f1b[...]), f2w[...]) + f2b[...]
    xd2_ref[...] = _ln(xd1 + ff, n2g[...], n2be[...])


def _global_attn(xd, qkv_w, qkv_b, w_post):
    qkvw3 = qkv_w.reshape(D, 3, H, DH).transpose(1, 2, 0, 3).reshape(
        3 * H, D, DH)
    qkvb3 = qkv_b.reshape(3, H, 1, DH).reshape(3 * H, 1, DH)
    wspec = pl.BlockSpec((1, D, DH), lambda h: (h, 0, 0))
    bspec = pl.BlockSpec((1, 1, DH), lambda h: (h, 0, 0))
    og3 = pl.pallas_call(
        _ga_attn_body,
        grid=(H,),
        in_specs=[_full((M, D)),
                  pl.BlockSpec((1, D, DH), lambda h: (h, 0, 0)),
                  pl.BlockSpec((1, 1, DH), lambda h: (h, 0, 0)),
                  pl.BlockSpec((1, D, DH), lambda h: (H + h, 0, 0)),
                  pl.BlockSpec((1, 1, DH), lambda h: (H + h, 0, 0)),
                  pl.BlockSpec((1, D, DH), lambda h: (2 * H + h, 0, 0)),
                  pl.BlockSpec((1, 1, DH), lambda h: (2 * H + h, 0, 0))],
        out_specs=pl.BlockSpec((1, M, DH), lambda h: (h, 0, 0)),
        out_shape=jax.ShapeDtypeStruct((H, M, DH), f32),
    )(xd, qkvw3, qkvb3, qkvw3, qkvb3, qkvw3, qkvb3)
    gow3 = w_post[0].reshape(H, DH, D)
    return pl.pallas_call(
        _ga_post_body,
        grid=(1,),
        in_specs=[_full((M, D)), _full((H, M, DH)), _full((H, DH, D))] +
                 [_full(a.shape) for a in w_post[1:]],
        out_specs=_full((M, D)),
        out_shape=jax.ShapeDtypeStruct((M, D), f32),
    )(xd, og3, gow3, *w_post[1:])


def _fp_body(xdg_ref, x2_ref, w1a, w1b, b1, g1, be1, w2, b2, g2, be2, y_ref):
    cat = _mm(xdg_ref[...], w1a[...]) + _mm(x2_ref[...], w1b[...]) + b1[...]
    y = jax.nn.gelu(_ln(cat, g1[...], be1[...]))
    y = jax.nn.gelu(_ln(_mm(y, w2[...]) + b2[...], g2[...], be2[...]))
    y_ref[...] = y


def _fprop(xdg, x2, w):
    R = 512
    in_arrs = [xdg, x2] + w
    in_specs = ([_rows((R, D)), _rows((R, D))] + [_full(a.shape) for a in w])
    return pl.pallas_call(
        _fp_body,
        grid=(N // R,),
        in_specs=in_specs,
        out_specs=_rows((R, D)),
        out_shape=jax.ShapeDtypeStruct((N, D), f32),
    )(*in_arrs)


def kernel(pos, feat, params):
    p = params
    row = lambda a: a.reshape(1, -1)

    posp128 = jnp.pad(pos, ((0, 0), (0, 125)))
    pos8 = posp128[:, :8]
    post8 = pos8.T
    sq = jnp.sum(pos * pos, -1)
    sqr = sq.reshape(N, 1)
    sqc = sq.reshape(1, N)

    emb_w = [p['ce_w1'], row(p['ce_b1']), row(p['ce_g']), row(p['ce_be']),
             p['ce_w2'], row(p['ce_b2']),
             p['fe_w1'], row(p['fe_b1']), row(p['fe_g']), row(p['fe_be']),
             p['fe_w2'], row(p['fe_b2']),
             p['fu_w'][:D], p['fu_w'][D:], row(p['fu_b']), row(p['fu_g']),
             row(p['fu_be']),
             p['q_w'], row(p['q_b']), p['k_w'], row(p['k_b']),
             p['v_w'], row(p['v_b'])]
    x, qp, kp, vp = _embed(pos, feat, emb_w)

    knn = _knn(pos8, post8, sqr, sqc)
    idx = knn[:, :K]
    up = knn[:, K]

    idxf = idx.reshape(N * K)
    half = N * K // 2
    kg1, vg1, pg1 = _gather_multi([kp, vp, posp128], idxf[:half])
    kg2, vg2, pg2 = _gather_multi([kp, vp, posp128], idxf[half:])

    rp_w1p = jnp.pad(p['rp_w1'], ((0, 125), (0, 0)))
    attn_w = [rp_w1p, row(p['rp_b1']), p['rp_w2'], row(p['rp_b2']),
              p['o_w'], row(p['o_b']), row(p['la_g']), row(p['la_be'])]
    x2a = _local_attn(qp, kg1, vg1, pg1, posp128, x, attn_w, 0)
    x2b = _local_attn(qp, kg2, vg2, pg2, posp128, x, attn_w, N // 2)
    x2 = jnp.concatenate([x2a, x2b], axis=0)

    gi = idx[::4].reshape(M * K)
    xg, pga = _gather_multi([x2, posp128], gi)
    posa16 = posp128[::4]
    saw_p = jnp.pad(p['sa_w'][D:], ((0, 125), (0, 0)))
    sa_w = [p['sa_w'][:D], saw_p, row(p['sa_b']), row(p['sa_g']),
            row(p['sa_be'])]
    xd = _set_abs(xg, pga, posa16, sa_w)

    ga_post = [p['go_w'], row(p['go_b']), row(p['n1_g']), row(p['n1_be']),
               p['f1_w'], row(p['f1_b']), p['f2_w'], row(p['f2_b']),
               row(p['n2_g']), row(p['n2_be'])]
    xd2 = _global_attn(xd, p['qkv_w'], p['qkv_b'], ga_post)

    xdg = _gather_rows(xd2, up)

    fp_w = [p['fp_w1'][:D], p['fp_w1'][D:], row(p['fp_b1']), row(p['fp_g1']),
            row(p['fp_be1']), p['fp_w2'], row(p['fp_b2']), row(p['fp_g2']),
            row(p['fp_be2'])]
    return _fprop(xdg, x2, fp_w)


# speedup vs baseline: 1.2210x; 1.2210x over previous
import functools

import numpy as np

import jax
import jax.numpy as jnp
from jax import lax
from jax.experimental import pallas as pl
from jax.experimental.pallas import tpu as pltpu
from jax.experimental.pallas import tpu_sc as plsc

N = 8192
CIN = 6
D = 256
H = 8
DH = D // H
K = 16
M = N // 4
HID = 64
SCALE = DH ** -0.5

bf16 = jnp.bfloat16
f32 = jnp.float32
i32 = jnp.int32


def _mm(a, b, prec=None):
    if prec is None:
        a = a.astype(bf16)
        b = b.astype(bf16)
    return lax.dot_general(a, b, (((a.ndim - 1,), (0,)), ((), ())),
                           preferred_element_type=f32,
                           precision=prec)


def _ln(x, g, b):
    mu = jnp.mean(x, -1, keepdims=True)
    v = jnp.mean((x - mu) ** 2, -1, keepdims=True)
    return g * (x - mu) / jnp.sqrt(v + 1e-5) + b


def _full(shape):
    nd = len(shape)
    return pl.BlockSpec(shape, lambda i: (0,) * nd)


def _rows(bshape):
    nd = len(bshape)
    return pl.BlockSpec(bshape, lambda i: (i,) + (0,) * (nd - 1))


def _embed_body(pos_ref, feat_ref,
                cw1, cb1, cg, cbe, cw2, cb2,
                fw1, fb1, fg, fbe, fw2, fb2,
                fuwa, fuwb, fub, fug, fube,
                qw, qb, kw, kb, vw, vb,
                x_ref, q_ref, k_ref, v_ref):
    pe = _mm(jax.nn.gelu(_ln(_mm(pos_ref[...], cw1[...]) + cb1[...],
                             cg[...], cbe[...])), cw2[...]) + cb2[...]
    fe = _mm(jax.nn.gelu(_ln(_mm(feat_ref[...], fw1[...]) + fb1[...],
                             fg[...], fbe[...])), fw2[...]) + fb2[...]
    fu = _mm(pe, fuwa[...]) + _mm(fe, fuwb[...]) + fub[...]
    x = jax.nn.gelu(_ln(fu, fug[...], fube[...]))
    x_ref[...] = x
    q_ref[...] = _mm(x, qw[...]) + qb[...]
    k_ref[...] = _mm(x, kw[...]) + kb[...]
    v_ref[...] = _mm(x, vw[...]) + vb[...]


def _embed(pos, feat, w):
    R = 512
    outs = [jax.ShapeDtypeStruct((N, D), f32)] * 4
    in_arrs = [pos, feat] + w
    in_specs = [_rows((R, 3)), _rows((R, CIN))] + [_full(a.shape) for a in w]
    return pl.pallas_call(
        _embed_body,
        grid=(N // R,),
        in_specs=in_specs,
        out_specs=[_rows((R, D))] * 4,
        out_shape=outs,
    )(*in_arrs)


_RK = 64
_CH = 1024
_NCH = N // _CH
_NL = 8
_INF = np.int32(0x7FFFFFFF)
_BIGP = np.int32(1 << 30)


def _knn_body(pos_ref, post_ref, sqr_ref, sqc_ref, pta_ref, sqa_ref,
              out_ref):
    pos_b = pos_ref[...].astype(bf16)
    sqr = sqr_ref[...]
    lane = lax.broadcasted_iota(i32, (_RK, _CH), 1)
    g_local = lane >> 7

    def chunk(c, lists):
        lists = list(lists)
        off = pl.multiple_of(c * _CH, _CH)
        ptc = post_ref[:, pl.ds(off, _CH)].astype(bf16)
        d = sqr + sqc_ref[:, pl.ds(off, _CH)] - 2.0 * lax.dot_general(
            pos_b, ptc, (((1,), (0,)), ((), ())), preferred_element_type=f32)
        b = lax.bitcast_convert_type(d + 0.5, i32)
        keys = lax.bitcast_convert_type(
            (b & jnp.int32(-64)) | (g_local + c * (_CH // 128)), f32)
        for s in range(_CH // 128):
            kg = keys[:, s * 128:(s + 1) * 128]
            for j in range(_NL):
                lo = jnp.minimum(lists[j], kg)
                kg = jnp.maximum(lists[j], kg)
                lists[j] = lo
        return tuple(lists)

    init = tuple(jnp.full((_RK, 128), jnp.inf, f32) for _ in range(_NL))
    lists = lax.fori_loop(0, _NCH, chunk, init)

    da = sqr + sqa_ref[...] - 2.0 * lax.dot_general(
        pos_b, pta_ref[...].astype(bf16), (((1,), (0,)), ((), ())),
        preferred_element_type=f32)
    dmin = jnp.full((_RK, 128), jnp.inf, f32)
    gmin = jnp.zeros((_RK, 128), i32)
    for s in range(M // 128):
        ds_ = da[:, s * 128:(s + 1) * 128]
        cond = ds_ < dmin
        gmin = jnp.where(cond, s, gmin)
        dmin = jnp.minimum(dmin, ds_)
    lane128 = lax.broadcasted_iota(i32, (_RK, 128), 1)
    mu_ = jnp.min(dmin, axis=1, keepdims=True)
    up_col = jnp.min(jnp.where(dmin == mu_, gmin * 128 + lane128, _BIGP),
                     axis=1, keepdims=True)

    cand = jnp.concatenate(lists, axis=1)
    lane_c = lax.broadcasted_iota(i32, (_RK, _NL * 128), 1)
    acc = jnp.zeros((_RK, 24), i32)
    kio = lax.broadcasted_iota(i32, (_RK, 24), 1)
    for kk in range(K):
        m = jnp.min(cand, axis=1, keepdims=True)
        p = jnp.min(jnp.where(cand == m, lane_c, _BIGP), axis=1, keepdims=True)
        col = (lax.bitcast_convert_type(m, i32) & 63) * 128 + (p & 127)
        acc = jnp.where(kio == kk, col, acc)
        cand = jnp.where(lane_c == p, jnp.inf, cand)
    acc = jnp.where(kio == K, up_col, acc)
    out_ref[...] = acc


def _knn(pos8, post8, sqr, sqc, pta, sqa):
    return pl.pallas_call(
        _knn_body,
        grid=(N // _RK,),
        in_specs=[_rows((_RK, 8)), _full((8, N)), _rows((_RK, 1)),
                  _full((1, N)), _full((8, M)), _full((1, M))],
        out_specs=_rows((_RK, 24)),
        out_shape=jax.ShapeDtypeStruct((N, 24), i32),
    )(pos8, post8, sqr, sqc, pta, sqa)


_NW = 32


def _gather_multi(tables, idx):
    B = idx.shape[0]
    T = len(tables)
    bw = B // _NW
    wsum = sum(t.shape[1] for t in tables)
    CH = 128 if wsum <= 384 else 64
    nch = bw // CH
    idx2d = idx.reshape(B // CH, CH)
    mesh = plsc.VectorSubcoreMesh(core_axis_name="c", subcore_axis_name="s")

    scratch = [pltpu.VMEM((nch, CH), i32)]
    for t in tables:
        scratch += [pltpu.VMEM((CH, t.shape[1]), t.dtype)] * 2
    scratch += [pltpu.SemaphoreType.DMA] * (4 * T)

    @functools.partial(
        pl.kernel,
        out_type=[jax.ShapeDtypeStruct((B, t.shape[1]), t.dtype)
                  for t in tables],
        mesh=mesh,
        scratch_types=scratch,
    )
    def gk(*refs):
        tabs = refs[:T]
        idx_hbm = refs[T]
        outs = refs[T + 1:2 * T + 1]
        idx_v = refs[2 * T + 1]
        bufs = refs[2 * T + 2:2 * T + 2 + 2 * T]
        sems = refs[2 * T + 2 + 2 * T:]
        rows = [(bufs[2 * i], bufs[2 * i + 1]) for i in range(T)]
        sg = [(sems[4 * i], sems[4 * i + 1]) for i in range(T)]
        ss = [(sems[4 * i + 2], sems[4 * i + 3]) for i in range(T)]
        wid = lax.axis_index("s") * 2 + lax.axis_index("c")
        base = wid * bw
        pltpu.sync_copy(idx_hbm.at[pl.ds(wid * nch, nch)], idx_v)
        for t in range(T):
            pltpu.async_copy(tabs[t].at[idx_v.at[0]], rows[t][0], sg[t][0])

        def body(c2, carry):
            c = 2 * c2
            for t in range(T):
                pltpu.make_async_copy(tabs[t].at[idx_v.at[c]], rows[t][0],
                                      sg[t][0]).wait()

            @pl.when(c2 > 0)
            def _():
                for t in range(T):
                    pltpu.make_async_copy(rows[t][1],
                                          outs[t].at[pl.ds(base, CH)],
                                          ss[t][1]).wait()

            for t in range(T):
                pltpu.async_copy(tabs[t].at[idx_v.at[c + 1]], rows[t][1],
                                 sg[t][1])
            for t in range(T):
                pltpu.async_copy(rows[t][0],
                                 outs[t].at[pl.ds(base + c * CH, CH)],
                                 ss[t][0])
            for t in range(T):
                pltpu.make_async_copy(tabs[t].at[idx_v.at[c + 1]], rows[t][1],
                                      sg[t][1]).wait()

            @pl.when(c2 < nch // 2 - 1)
            def _():
                for t in range(T):
                    pltpu.make_async_copy(rows[t][0],
                                          outs[t].at[pl.ds(base, CH)],
                                          ss[t][0]).wait()
                    pltpu.async_copy(tabs[t].at[idx_v.at[c + 2]], rows[t][0],
                                     sg[t][0])

            for t in range(T):
                pltpu.async_copy(rows[t][1],
                                 outs[t].at[pl.ds(base + (c + 1) * CH, CH)],
                                 ss[t][1])
            return carry

        lax.fori_loop(0, nch // 2, body, 0)
        for t in range(T):
            pltpu.make_async_copy(rows[t][0], outs[t].at[pl.ds(base, CH)],
                                  ss[t][0]).wait()
            pltpu.make_async_copy(rows[t][1], outs[t].at[pl.ds(base, CH)],
                                  ss[t][1]).wait()

    out = gk(*tables, idx2d)
    return out if isinstance(out, (list, tuple)) else [out]


def _gather_rows(table, idx):
    return _gather_multi([table], idx)[0]


_RA = 128


def _attn_body(q_ref, kg_ref, vg_ref, pg_ref, posp_ref, x_ref,
               rw1, rb1, rw2, rb2, ow, ob, lag, labe, s_ref, st_ref,
               x2_ref):
    RK = _RA * K
    pos_rep = jnp.broadcast_to(posp_ref[...][:, None, :],
                               (_RA, K, 128)).reshape(RK, 128)
    rel = pg_ref[...] - pos_rep
    bias = _mm(jax.nn.gelu(_mm(rel, rw1[...]) + rb1[...]), rw2[...]) + rb2[...]

    q_rep = jnp.broadcast_to(q_ref[...][:, None, :],
                             (_RA, K, D)).reshape(RK, D)
    qb = q_rep.astype(bf16).astype(f32)
    kb = kg_ref[...].astype(bf16).astype(f32)
    prod = qb * kb
    logits = _mm(prod, s_ref[...], prec=lax.Precision.HIGHEST) * SCALE + bias
    l3 = logits.reshape(_RA, K, H)
    mx = jnp.max(l3, axis=1, keepdims=True)
    e = jnp.exp(l3 - mx)
    sm = (e / jnp.sum(e, axis=1, keepdims=True)).reshape(RK, H)
    a_exp = _mm(sm.astype(bf16).astype(f32), st_ref[...],
                prec=lax.Precision.HIGHEST)
    vb = vg_ref[...].astype(bf16).astype(f32)
    o = jnp.sum((a_exp * vb).reshape(_RA, K, D), axis=1)
    out = _mm(o, ow[...]) + ob[...]
    x2_ref[...] = _ln(x_ref[...] + out, lag[...], labe[...])


def _local_attn(qp, kg, vg, pg, posp16, x, w, off):
    smat = jnp.repeat(jnp.eye(H, dtype=f32), DH, axis=0)
    stmat = smat.T
    nrows = kg.shape[0] // K
    ob = off // _RA
    offrows = lambda bshape: pl.BlockSpec(
        bshape, lambda i: (i + ob,) + (0,) * (len(bshape) - 1))
    in_arrs = [qp, kg, vg, pg, posp16, x] + w + [smat, stmat]
    in_specs = ([offrows((_RA, D)), _rows((_RA * K, D)), _rows((_RA * K, D)),
                 _rows((_RA * K, 128)), offrows((_RA, 128)),
                 offrows((_RA, D))] +
                [_full(a.shape) for a in w] +
                [_full((D, H)), _full((H, D))])
    return pl.pallas_call(
        _attn_body,
        grid=(nrows // _RA,),
        in_specs=in_specs,
        out_specs=_rows((_RA, D)),
        out_shape=jax.ShapeDtypeStruct((nrows, D), f32),
    )(*in_arrs)


def _sa_body(xg_ref, pga_ref, posa_ref, swx, swp, sb, sg, sbe, xd_ref):
    RK = _RA * K
    pos_rep = jnp.broadcast_to(posa_ref[...][:, None, :],
                               (_RA, K, 128)).reshape(RK, 128)
    rel = pga_ref[...] - pos_rep
    gin = _mm(xg_ref[...], swx[...]) + _mm(rel, swp[...]) + sb[...]
    g = jax.nn.gelu(_ln(gin, sg[...], sbe[...]))
    xd_ref[...] = jnp.max(g.reshape(_RA, K, D), axis=1)


def _set_abs(xg, pga, posa16, w):
    in_arrs = [xg, pga, posa16] + w
    in_specs = ([_rows((_RA * K, D)), _rows((_RA * K, 128)),
                 _rows((_RA, 128))] + [_full(a.shape) for a in w])
    return pl.pallas_call(
        _sa_body,
        grid=(M // _RA,),
        in_specs=in_specs,
        out_specs=_rows((_RA, D)),
        out_shape=jax.ShapeDtypeStruct((M, D), f32),
    )(*in_arrs)


def _ga_attn_body(xd_ref, qw, qb, kw, kb, vw, vb, og_ref):
    qh = (_mm(xd_ref[...], qw[0]) + qb[0]).astype(bf16)
    kh = (_mm(xd_ref[...], kw[0]) + kb[0]).astype(bf16)
    vh = _mm(xd_ref[...], vw[0]) + vb[0]
    s = lax.dot_general(qh, kh, (((1,), (1,)), ((), ())),
                        preferred_element_type=f32) * SCALE
    mx = jnp.max(s, axis=1, keepdims=True)
    e = jnp.exp(s - mx)
    a = e / jnp.sum(e, axis=1, keepdims=True)
    og_ref[0] = _mm(a, vh)


def _ga_post_body(xd_ref, og_ref, gow, gob, n1g, n1be, f1w, f1b, f2w, f2b,
                  n2g, n2be, xd2_ref):
    og = gob[...]
    for h in range(H):
        og = og + _mm(og_ref[h], gow[h])
    xd1 = _ln(xd_ref[...] + og, n1g[...], n1be[...])
    ff = _mm(jax.nn.gelu(_mm(xd1, f1w[...]) + f1b[...]), f2w[...]) + f2b[...]
    xd2_ref[...] = _ln(xd1 + ff, n2g[...], n2be[...])


def _global_attn(xd, qkv_w, qkv_b, w_post):
    qkvw3 = qkv_w.reshape(D, 3, H, DH).transpose(1, 2, 0, 3).reshape(
        3 * H, D, DH)
    qkvb3 = qkv_b.reshape(3, H, 1, DH).reshape(3 * H, 1, DH)
    wspec = pl.BlockSpec((1, D, DH), lambda h: (h, 0, 0))
    bspec = pl.BlockSpec((1, 1, DH), lambda h: (h, 0, 0))
    og3 = pl.pallas_call(
        _ga_attn_body,
        grid=(H,),
        in_specs=[_full((M, D)),
                  pl.BlockSpec((1, D, DH), lambda h: (h, 0, 0)),
                  pl.BlockSpec((1, 1, DH), lambda h: (h, 0, 0)),
                  pl.BlockSpec((1, D, DH), lambda h: (H + h, 0, 0)),
                  pl.BlockSpec((1, 1, DH), lambda h: (H + h, 0, 0)),
                  pl.BlockSpec((1, D, DH), lambda h: (2 * H + h, 0, 0)),
                  pl.BlockSpec((1, 1, DH), lambda h: (2 * H + h, 0, 0))],
        out_specs=pl.BlockSpec((1, M, DH), lambda h: (h, 0, 0)),
        out_shape=jax.ShapeDtypeStruct((H, M, DH), f32),
    )(xd, qkvw3, qkvb3, qkvw3, qkvb3, qkvw3, qkvb3)
    gow3 = w_post[0].reshape(H, DH, D)
    return pl.pallas_call(
        _ga_post_body,
        grid=(1,),
        in_specs=[_full((M, D)), _full((H, M, DH)), _full((H, DH, D))] +
                 [_full(a.shape) for a in w_post[1:]],
        out_specs=_full((M, D)),
        out_shape=jax.ShapeDtypeStruct((M, D), f32),
    )(xd, og3, gow3, *w_post[1:])


def _fp_body(xdg_ref, x2_ref, w1a, w1b, b1, g1, be1, w2, b2, g2, be2, y_ref):
    cat = _mm(xdg_ref[...], w1a[...]) + _mm(x2_ref[...], w1b[...]) + b1[...]
    y = jax.nn.gelu(_ln(cat, g1[...], be1[...]))
    y = jax.nn.gelu(_ln(_mm(y, w2[...]) + b2[...], g2[...], be2[...]))
    y_ref[...] = y


def _fprop(xdg, x2, w):
    R = 512
    in_arrs = [xdg, x2] + w
    in_specs = ([_rows((R, D)), _rows((R, D))] + [_full(a.shape) for a in w])
    return pl.pallas_call(
        _fp_body,
        grid=(N // R,),
        in_specs=in_specs,
        out_specs=_rows((R, D)),
        out_shape=jax.ShapeDtypeStruct((N, D), f32),
    )(*in_arrs)


def kernel(pos, feat, params):
    p = params
    row = lambda a: a.reshape(1, -1)

    posp128 = jnp.pad(pos, ((0, 0), (0, 125)))
    pos8 = posp128[:, :8]
    post8 = pos8.T
    sq = jnp.sum(pos * pos, -1)
    sqr = sq.reshape(N, 1)
    sqc = sq.reshape(1, N)

    emb_w = [p['ce_w1'], row(p['ce_b1']), row(p['ce_g']), row(p['ce_be']),
             p['ce_w2'], row(p['ce_b2']),
             p['fe_w1'], row(p['fe_b1']), row(p['fe_g']), row(p['fe_be']),
             p['fe_w2'], row(p['fe_b2']),
             p['fu_w'][:D], p['fu_w'][D:], row(p['fu_b']), row(p['fu_g']),
             row(p['fu_be']),
             p['q_w'], row(p['q_b']), p['k_w'], row(p['k_b']),
             p['v_w'], row(p['v_b'])]
    x, qp, kp, vp = _embed(pos, feat, emb_w)

    knn = _knn(pos8, post8, sqr, sqc, post8[:, ::4], sqc[:, ::4])
    idx = knn[:, :K]
    up = knn[:, K]

    idxf = idx.reshape(N * K)
    half = N * K // 2
    kg1, vg1, pg1 = _gather_multi([kp, vp, posp128], idxf[:half])
    kg2, vg2, pg2 = _gather_multi([kp, vp, posp128], idxf[half:])

    rp_w1p = jnp.pad(p['rp_w1'], ((0, 125), (0, 0)))
    attn_w = [rp_w1p, row(p['rp_b1']), p['rp_w2'], row(p['rp_b2']),
              p['o_w'], row(p['o_b']), row(p['la_g']), row(p['la_be'])]
    x2a = _local_attn(qp, kg1, vg1, pg1, posp128, x, attn_w, 0)
    x2b = _local_attn(qp, kg2, vg2, pg2, posp128, x, attn_w, N // 2)
    x2 = jnp.concatenate([x2a, x2b], axis=0)

    gi = idx[::4].reshape(M * K)
    xg, pga = _gather_multi([x2, posp128], gi)
    posa16 = posp128[::4]
    saw_p = jnp.pad(p['sa_w'][D:], ((0, 125), (0, 0)))
    sa_w = [p['sa_w'][:D], saw_p, row(p['sa_b']), row(p['sa_g']),
            row(p['sa_be'])]
    xd = _set_abs(xg, pga, posa16, sa_w)

    ga_post = [p['go_w'], row(p['go_b']), row(p['n1_g']), row(p['n1_be']),
               p['f1_w'], row(p['f1_b']), p['f2_w'], row(p['f2_b']),
               row(p['n2_g']), row(p['n2_be'])]
    xd2 = _global_attn(xd, p['qkv_w'], p['qkv_b'], ga_post)

    xdg = _gather_rows(xd2, up)

    fp_w = [p['fp_w1'][:D], p['fp_w1'][D:], row(p['fp_b1']), row(p['fp_g1']),
            row(p['fp_be1']), p['fp_w2'], row(p['fp_b2']), row(p['fp_g2']),
            row(p['fp_be2'])]
    return _fprop(xdg, x2, fp_w)


# speedup vs baseline: 1.2513x; 1.0248x over previous
import functools

import numpy as np

import jax
import jax.numpy as jnp
from jax import lax
from jax.experimental import pallas as pl
from jax.experimental.pallas import tpu as pltpu
from jax.experimental.pallas import tpu_sc as plsc

N = 8192
CIN = 6
D = 256
H = 8
DH = D // H
K = 16
M = N // 4
HID = 64
SCALE = DH ** -0.5

bf16 = jnp.bfloat16
f32 = jnp.float32
i32 = jnp.int32


def _mm(a, b, prec=None):
    if prec is None:
        a = a.astype(bf16)
        b = b.astype(bf16)
    return lax.dot_general(a, b, (((a.ndim - 1,), (0,)), ((), ())),
                           preferred_element_type=f32,
                           precision=prec)


def _ln(x, g, b):
    mu = jnp.mean(x, -1, keepdims=True)
    v = jnp.mean((x - mu) ** 2, -1, keepdims=True)
    return g * (x - mu) / jnp.sqrt(v + 1e-5) + b


def _full(shape):
    nd = len(shape)
    return pl.BlockSpec(shape, lambda i: (0,) * nd)


def _rows(bshape):
    nd = len(bshape)
    return pl.BlockSpec(bshape, lambda i: (i,) + (0,) * (nd - 1))


def _embed_body(pos_ref, feat_ref,
                cw1, cb1, cg, cbe, cw2, cb2,
                fw1, fb1, fg, fbe, fw2, fb2,
                fuwa, fuwb, fub, fug, fube,
                qw, qb, kw, kb, vw, vb,
                x_ref, q_ref, k_ref, v_ref):
    pe = _mm(jax.nn.gelu(_ln(_mm(pos_ref[...], cw1[...]) + cb1[...],
                             cg[...], cbe[...])), cw2[...]) + cb2[...]
    fe = _mm(jax.nn.gelu(_ln(_mm(feat_ref[...], fw1[...]) + fb1[...],
                             fg[...], fbe[...])), fw2[...]) + fb2[...]
    fu = _mm(pe, fuwa[...]) + _mm(fe, fuwb[...]) + fub[...]
    x = jax.nn.gelu(_ln(fu, fug[...], fube[...]))
    x_ref[...] = x
    q_ref[...] = _mm(x, qw[...]) + qb[...]
    k_ref[...] = _mm(x, kw[...]) + kb[...]
    v_ref[...] = _mm(x, vw[...]) + vb[...]


def _embed(pos, feat, w):
    R = 512
    outs = [jax.ShapeDtypeStruct((N, D), f32)] * 4
    in_arrs = [pos, feat] + w
    in_specs = [_rows((R, 3)), _rows((R, CIN))] + [_full(a.shape) for a in w]
    return pl.pallas_call(
        _embed_body,
        grid=(N // R,),
        in_specs=in_specs,
        out_specs=[_rows((R, D))] * 4,
        out_shape=outs,
    )(*in_arrs)


_RK = 64
_CH = 1024
_NCH = N // _CH
_NL = 8
_INF = np.int32(0x7FFFFFFF)
_BIGP = np.int32(1 << 30)


def _knn_body(pos_ref, post_ref, sqr_ref, sqc_ref, pta_ref, sqa_ref,
              out_ref):
    pos_b = pos_ref[...].astype(bf16)
    sqr = sqr_ref[...]
    lane = lax.broadcasted_iota(i32, (_RK, _CH), 1)
    g_local = lane >> 7

    def chunk(c, lists):
        lists = list(lists)
        off = pl.multiple_of(c * _CH, _CH)
        ptc = post_ref[:, pl.ds(off, _CH)].astype(bf16)
        d = sqr + sqc_ref[:, pl.ds(off, _CH)] - 2.0 * lax.dot_general(
            pos_b, ptc, (((1,), (0,)), ((), ())), preferred_element_type=f32)
        b = lax.bitcast_convert_type(d + 0.5, i32)
        keys = lax.bitcast_convert_type(
            (b & jnp.int32(-64)) | (g_local + c * (_CH // 128)), f32)
        for s in range(_CH // 128):
            kg = keys[:, s * 128:(s + 1) * 128]
            for j in range(_NL):
                lo = jnp.minimum(lists[j], kg)
                kg = jnp.maximum(lists[j], kg)
                lists[j] = lo
        return tuple(lists)

    init = tuple(jnp.full((_RK, 128), jnp.inf, f32) for _ in range(_NL))
    lists = lax.fori_loop(0, _NCH, chunk, init)

    da = sqr + sqa_ref[...] - 2.0 * lax.dot_general(
        pos_b, pta_ref[...].astype(bf16), (((1,), (0,)), ((), ())),
        preferred_element_type=f32)
    dmin = jnp.full((_RK, 128), jnp.inf, f32)
    gmin = jnp.zeros((_RK, 128), i32)
    for s in range(M // 128):
        ds_ = da[:, s * 128:(s + 1) * 128]
        cond = ds_ < dmin
        gmin = jnp.where(cond, s, gmin)
        dmin = jnp.minimum(dmin, ds_)
    lane128 = lax.broadcasted_iota(i32, (_RK, 128), 1)
    mu_ = jnp.min(dmin, axis=1, keepdims=True)
    up_col = jnp.min(jnp.where(dmin == mu_, gmin * 128 + lane128, _BIGP),
                     axis=1, keepdims=True)

    cand = jnp.concatenate(lists, axis=1)
    lane_c = lax.broadcasted_iota(i32, (_RK, _NL * 128), 1)
    acc = jnp.zeros((_RK, 24), i32)
    kio = lax.broadcasted_iota(i32, (_RK, 24), 1)
    for kk in range(K):
        m = jnp.min(cand, axis=1, keepdims=True)
        p = jnp.min(jnp.where(cand == m, lane_c, _BIGP), axis=1, keepdims=True)
        col = (lax.bitcast_convert_type(m, i32) & 63) * 128 + (p & 127)
        acc = jnp.where(kio == kk, col, acc)
        cand = jnp.where(lane_c == p, jnp.inf, cand)
    acc = jnp.where(kio == K, up_col, acc)
    out_ref[...] = acc


def _knn(pos8, post8, sqr, sqc, pta, sqa):
    return pl.pallas_call(
        _knn_body,
        grid=(N // _RK,),
        in_specs=[_rows((_RK, 8)), _full((8, N)), _rows((_RK, 1)),
                  _full((1, N)), _full((8, M)), _full((1, M))],
        out_specs=_rows((_RK, 24)),
        out_shape=jax.ShapeDtypeStruct((N, 24), i32),
    )(pos8, post8, sqr, sqc, pta, sqa)


_NW = 32


def _gather_multi(tables, idx):
    B = idx.shape[0]
    T = len(tables)
    bw = B // _NW
    wsum = sum(t.shape[1] for t in tables)
    CH = 128 if wsum <= 384 else 64
    nch = bw // CH
    idx2d = idx.reshape(B // CH, CH)
    mesh = plsc.VectorSubcoreMesh(core_axis_name="c", subcore_axis_name="s")

    scratch = [pltpu.VMEM((nch, CH), i32)]
    for t in tables:
        scratch += [pltpu.VMEM((CH, t.shape[1]), t.dtype)] * 2
    scratch += [pltpu.SemaphoreType.DMA] * (4 * T)

    @functools.partial(
        pl.kernel,
        out_type=[jax.ShapeDtypeStruct((B, t.shape[1]), t.dtype)
                  for t in tables],
        mesh=mesh,
        scratch_types=scratch,
    )
    def gk(*refs):
        tabs = refs[:T]
        idx_hbm = refs[T]
        outs = refs[T + 1:2 * T + 1]
        idx_v = refs[2 * T + 1]
        bufs = refs[2 * T + 2:2 * T + 2 + 2 * T]
        sems = refs[2 * T + 2 + 2 * T:]
        rows = [(bufs[2 * i], bufs[2 * i + 1]) for i in range(T)]
        sg = [(sems[4 * i], sems[4 * i + 1]) for i in range(T)]
        ss = [(sems[4 * i + 2], sems[4 * i + 3]) for i in range(T)]
        wid = lax.axis_index("s") * 2 + lax.axis_index("c")
        base = wid * bw
        pltpu.sync_copy(idx_hbm.at[pl.ds(wid * nch, nch)], idx_v)
        for t in range(T):
            pltpu.async_copy(tabs[t].at[idx_v.at[0]], rows[t][0], sg[t][0])

        def body(c2, carry):
            c = 2 * c2
            for t in range(T):
                pltpu.make_async_copy(tabs[t].at[idx_v.at[c]], rows[t][0],
                                      sg[t][0]).wait()

            @pl.when(c2 > 0)
            def _():
                for t in range(T):
                    pltpu.make_async_copy(rows[t][1],
                                          outs[t].at[pl.ds(base, CH)],
                                          ss[t][1]).wait()

            for t in range(T):
                pltpu.async_copy(tabs[t].at[idx_v.at[c + 1]], rows[t][1],
                                 sg[t][1])
            for t in range(T):
                pltpu.async_copy(rows[t][0],
                                 outs[t].at[pl.ds(base + c * CH, CH)],
                                 ss[t][0])
            for t in range(T):
                pltpu.make_async_copy(tabs[t].at[idx_v.at[c + 1]], rows[t][1],
                                      sg[t][1]).wait()

            @pl.when(c2 < nch // 2 - 1)
            def _():
                for t in range(T):
                    pltpu.make_async_copy(rows[t][0],
                                          outs[t].at[pl.ds(base, CH)],
                                          ss[t][0]).wait()
                    pltpu.async_copy(tabs[t].at[idx_v.at[c + 2]], rows[t][0],
                                     sg[t][0])

            for t in range(T):
                pltpu.async_copy(rows[t][1],
                                 outs[t].at[pl.ds(base + (c + 1) * CH, CH)],
                                 ss[t][1])
            return carry

        lax.fori_loop(0, nch // 2, body, 0)
        for t in range(T):
            pltpu.make_async_copy(rows[t][0], outs[t].at[pl.ds(base, CH)],
                                  ss[t][0]).wait()
            pltpu.make_async_copy(rows[t][1], outs[t].at[pl.ds(base, CH)],
                                  ss[t][1]).wait()

    out = gk(*tables, idx2d)
    return out if isinstance(out, (list, tuple)) else [out]


def _gather_rows(table, idx):
    return _gather_multi([table], idx)[0]


_RA = 128


def _attn_body(q_ref, kg_ref, vg_ref, pg_ref, posp_ref, x_ref,
               rw1, rb1, rw2, rb2, ow, ob, lag, labe, s_ref, st_ref,
               x2_ref):
    RK = _RA * K
    pos_rep = jnp.broadcast_to(posp_ref[...][:, None, :],
                               (_RA, K, 128)).reshape(RK, 128)
    rel = pg_ref[...] - pos_rep
    bias = _mm(jax.nn.gelu(_mm(rel, rw1[...]) + rb1[...]), rw2[...]) + rb2[...]

    q_rep = jnp.broadcast_to(q_ref[...][:, None, :],
                             (_RA, K, D)).reshape(RK, D)
    qb = q_rep.astype(bf16).astype(f32)
    kb = kg_ref[...].astype(bf16).astype(f32)
    prod = qb * kb
    logits = _mm(prod, s_ref[...], prec=lax.Precision.HIGHEST) * SCALE + bias
    l3 = logits.reshape(_RA, K, H)
    mx = jnp.max(l3, axis=1, keepdims=True)
    e = jnp.exp(l3 - mx)
    sm = (e / jnp.sum(e, axis=1, keepdims=True)).reshape(RK, H)
    a_exp = _mm(sm.astype(bf16).astype(f32), st_ref[...],
                prec=lax.Precision.HIGHEST)
    vb = vg_ref[...].astype(bf16).astype(f32)
    o = jnp.sum((a_exp * vb).reshape(_RA, K, D), axis=1)
    out = _mm(o, ow[...]) + ob[...]
    x2_ref[...] = _ln(x_ref[...] + out, lag[...], labe[...])


def _local_attn(qp, kg, vg, pg, posp16, x, w, off):
    smat = jnp.repeat(jnp.eye(H, dtype=f32), DH, axis=0)
    stmat = smat.T
    nrows = kg.shape[0] // K
    ob = off // _RA
    offrows = lambda bshape: pl.BlockSpec(
        bshape, lambda i: (i + ob,) + (0,) * (len(bshape) - 1))
    in_arrs = [qp, kg, vg, pg, posp16, x] + w + [smat, stmat]
    in_specs = ([offrows((_RA, D)), _rows((_RA * K, D)), _rows((_RA * K, D)),
                 _rows((_RA * K, 128)), offrows((_RA, 128)),
                 offrows((_RA, D))] +
                [_full(a.shape) for a in w] +
                [_full((D, H)), _full((H, D))])
    return pl.pallas_call(
        _attn_body,
        grid=(nrows // _RA,),
        in_specs=in_specs,
        out_specs=_rows((_RA, D)),
        out_shape=jax.ShapeDtypeStruct((nrows, D), f32),
    )(*in_arrs)


def _sa_body(xg_ref, pga_ref, posa_ref, swx, swp, sb, sg, sbe, xd_ref):
    RK = _RA * K
    pos_rep = jnp.broadcast_to(posa_ref[...][:, None, :],
                               (_RA, K, 128)).reshape(RK, 128)
    rel = pga_ref[...] - pos_rep
    gin = _mm(xg_ref[...], swx[...]) + _mm(rel, swp[...]) + sb[...]
    g = jax.nn.gelu(_ln(gin, sg[...], sbe[...]))
    xd_ref[...] = jnp.max(g.reshape(_RA, K, D), axis=1)


def _set_abs(xg, pga, posa16, w):
    in_arrs = [xg, pga, posa16] + w
    in_specs = ([_rows((_RA * K, D)), _rows((_RA * K, 128)),
                 _rows((_RA, 128))] + [_full(a.shape) for a in w])
    return pl.pallas_call(
        _sa_body,
        grid=(M // _RA,),
        in_specs=in_specs,
        out_specs=_rows((_RA, D)),
        out_shape=jax.ShapeDtypeStruct((M, D), f32),
    )(*in_arrs)


def _ga_attn_body(xd_ref, qw, qb, kw, kb, vw, vb, og_ref):
    qh = (_mm(xd_ref[...], qw[0]) + qb[0]).astype(bf16)
    kh = (_mm(xd_ref[...], kw[0]) + kb[0]).astype(bf16)
    vh = _mm(xd_ref[...], vw[0]) + vb[0]
    s = lax.dot_general(qh, kh, (((1,), (1,)), ((), ())),
                        preferred_element_type=f32) * SCALE
    mx = jnp.max(s, axis=1, keepdims=True)
    e = jnp.exp(s - mx)
    a = e / jnp.sum(e, axis=1, keepdims=True)
    og_ref[0] = _mm(a, vh)


def _ga_post_body(xd_ref, og_ref, gow, gob, n1g, n1be, f1w, f1b, f2w, f2b,
                  n2g, n2be, xd2_ref):
    og = gob[...]
    for h in range(H):
        og = og + _mm(og_ref[h], gow[h])
    xd1 = _ln(xd_ref[...] + og, n1g[...], n1be[...])
    ff = _mm(jax.nn.gelu(_mm(xd1, f1w[...]) + f1b[...]), f2w[...]) + f2b[...]
    xd2_ref[...] = _ln(xd1 + ff, n2g[...], n2be[...])


def _global_attn(xd, qkv_w, qkv_b, w_post):
    qkvw3 = qkv_w.reshape(D, 3, H, DH).transpose(1, 2, 0, 3).reshape(
        3 * H, D, DH)
    qkvb3 = qkv_b.reshape(3, H, 1, DH).reshape(3 * H, 1, DH)
    wspec = pl.BlockSpec((1, D, DH), lambda h: (h, 0, 0))
    bspec = pl.BlockSpec((1, 1, DH), lambda h: (h, 0, 0))
    og3 = pl.pallas_call(
        _ga_attn_body,
        grid=(H,),
        in_specs=[_full((M, D)),
                  pl.BlockSpec((1, D, DH), lambda h: (h, 0, 0)),
                  pl.BlockSpec((1, 1, DH), lambda h: (h, 0, 0)),
                  pl.BlockSpec((1, D, DH), lambda h: (H + h, 0, 0)),
                  pl.BlockSpec((1, 1, DH), lambda h: (H + h, 0, 0)),
                  pl.BlockSpec((1, D, DH), lambda h: (2 * H + h, 0, 0)),
                  pl.BlockSpec((1, 1, DH), lambda h: (2 * H + h, 0, 0))],
        out_specs=pl.BlockSpec((1, M, DH), lambda h: (h, 0, 0)),
        out_shape=jax.ShapeDtypeStruct((H, M, DH), f32),
    )(xd, qkvw3, qkvb3, qkvw3, qkvb3, qkvw3, qkvb3)
    gow3 = w_post[0].reshape(H, DH, D)
    return pl.pallas_call(
        _ga_post_body,
        grid=(1,),
        in_specs=[_full((M, D)), _full((H, M, DH)), _full((H, DH, D))] +
                 [_full(a.shape) for a in w_post[1:]],
        out_specs=_full((M, D)),
        out_shape=jax.ShapeDtypeStruct((M, D), f32),
    )(xd, og3, gow3, *w_post[1:])


def _fp_body(xdg_ref, x2_ref, w1a, w1b, b1, g1, be1, w2, b2, g2, be2, y_ref):
    cat = _mm(xdg_ref[...], w1a[...]) + _mm(x2_ref[...], w1b[...]) + b1[...]
    y = jax.nn.gelu(_ln(cat, g1[...], be1[...]))
    y = jax.nn.gelu(_ln(_mm(y, w2[...]) + b2[...], g2[...], be2[...]))
    y_ref[...] = y


def _fprop(xdg, x2, w):
    R = 512
    in_arrs = [xdg, x2] + w
    in_specs = ([_rows((R, D)), _rows((R, D))] + [_full(a.shape) for a in w])
    return pl.pallas_call(
        _fp_body,
        grid=(N // R,),
        in_specs=in_specs,
        out_specs=_rows((R, D)),
        out_shape=jax.ShapeDtypeStruct((N, D), f32),
    )(*in_arrs)


def kernel(pos, feat, params):
    p = params
    row = lambda a: a.reshape(1, -1)

    posp128 = jnp.pad(pos, ((0, 0), (0, 125)))
    pos8 = posp128[:, :8]
    post8 = pos8.T
    sq = jnp.sum(pos * pos, -1)
    sqr = sq.reshape(N, 1)
    sqc = sq.reshape(1, N)

    emb_w = [p['ce_w1'], row(p['ce_b1']), row(p['ce_g']), row(p['ce_be']),
             p['ce_w2'], row(p['ce_b2']),
             p['fe_w1'], row(p['fe_b1']), row(p['fe_g']), row(p['fe_be']),
             p['fe_w2'], row(p['fe_b2']),
             p['fu_w'][:D], p['fu_w'][D:], row(p['fu_b']), row(p['fu_g']),
             row(p['fu_be']),
             p['q_w'], row(p['q_b']), p['k_w'], row(p['k_b']),
             p['v_w'], row(p['v_b'])]
    x, qp, kp, vp = _embed(pos, feat, emb_w)

    knn = _knn(pos8, post8, sqr, sqc, post8[:, ::4], sqc[:, ::4])
    idx = knn[:, :K]
    up = knn[:, K]

    idxf = idx.reshape(N * K)
    rp_w1p = jnp.pad(p['rp_w1'], ((0, 125), (0, 0)))
    attn_w = [rp_w1p, row(p['rp_b1']), p['rp_w2'], row(p['rp_b2']),
              p['o_w'], row(p['o_b']), row(p['la_g']), row(p['la_be'])]
    P = 4
    step = N * K // P
    parts = []
    for j in range(P):
        kgj, vgj, pgj = _gather_multi([kp, vp, posp128],
                                      idxf[j * step:(j + 1) * step])
        parts.append(_local_attn(qp, kgj, vgj, pgj, posp128, x, attn_w,
                                 j * (N // P)))
    x2 = jnp.concatenate(parts, axis=0)

    gi = idx[::4].reshape(M * K)
    xg, pga = _gather_multi([x2, posp128], gi)
    posa16 = posp128[::4]
    saw_p = jnp.pad(p['sa_w'][D:], ((0, 125), (0, 0)))
    sa_w = [p['sa_w'][:D], saw_p, row(p['sa_b']), row(p['sa_g']),
            row(p['sa_be'])]
    xd = _set_abs(xg, pga, posa16, sa_w)

    ga_post = [p['go_w'], row(p['go_b']), row(p['n1_g']), row(p['n1_be']),
               p['f1_w'], row(p['f1_b']), p['f2_w'], row(p['f2_b']),
               row(p['n2_g']), row(p['n2_be'])]
    xd2 = _global_attn(xd, p['qkv_w'], p['qkv_b'], ga_post)

    xdg = _gather_rows(xd2, up)

    fp_w = [p['fp_w1'][:D], p['fp_w1'][D:], row(p['fp_b1']), row(p['fp_g1']),
            row(p['fp_be1']), p['fp_w2'], row(p['fp_b2']), row(p['fp_g2']),
            row(p['fp_be2'])]
    return _fprop(xdg, x2, fp_w)


# speedup vs baseline: 1.5132x; 1.2093x over previous
import functools

import numpy as np

import jax
import jax.numpy as jnp
from jax import lax
from jax.experimental import pallas as pl
from jax.experimental.pallas import tpu as pltpu
from jax.experimental.pallas import tpu_sc as plsc

N = 8192
CIN = 6
D = 256
H = 8
DH = D // H
K = 16
M = N // 4
HID = 64
SCALE = DH ** -0.5

bf16 = jnp.bfloat16
f32 = jnp.float32
i32 = jnp.int32


def _mm(a, b, prec=None):
    if prec is None:
        a = a.astype(bf16)
        b = b.astype(bf16)
    return lax.dot_general(a, b, (((a.ndim - 1,), (0,)), ((), ())),
                           preferred_element_type=f32,
                           precision=prec)


def _ln(x, g, b):
    mu = jnp.mean(x, -1, keepdims=True)
    v = jnp.mean((x - mu) ** 2, -1, keepdims=True)
    return g * (x - mu) / jnp.sqrt(v + 1e-5) + b


def _full(shape):
    nd = len(shape)
    return pl.BlockSpec(shape, lambda i: (0,) * nd)


def _rows(bshape):
    nd = len(bshape)
    return pl.BlockSpec(bshape, lambda i: (i,) + (0,) * (nd - 1))


def _embed_body(pos_ref, feat_ref,
                cw1, cb1, cg, cbe, cw2, cb2,
                fw1, fb1, fg, fbe, fw2, fb2,
                fuwa, fuwb, fub, fug, fube,
                qw, qb, kw, kb, vw, vb,
                x_ref, q_ref, k_ref, v_ref):
    pe = _mm(jax.nn.gelu(_ln(_mm(pos_ref[...], cw1[...]) + cb1[...],
                             cg[...], cbe[...])), cw2[...]) + cb2[...]
    fe = _mm(jax.nn.gelu(_ln(_mm(feat_ref[...], fw1[...]) + fb1[...],
                             fg[...], fbe[...])), fw2[...]) + fb2[...]
    fu = _mm(pe, fuwa[...]) + _mm(fe, fuwb[...]) + fub[...]
    x = jax.nn.gelu(_ln(fu, fug[...], fube[...]))
    x_ref[...] = x
    q_ref[...] = _mm(x, qw[...]) + qb[...]
    k_ref[...] = _mm(x, kw[...]) + kb[...]
    v_ref[...] = _mm(x, vw[...]) + vb[...]


def _embed(pos, feat, w):
    R = 512
    outs = [jax.ShapeDtypeStruct((N, D), f32)] * 4
    in_arrs = [pos, feat] + w
    in_specs = [_rows((R, 3)), _rows((R, CIN))] + [_full(a.shape) for a in w]
    return pl.pallas_call(
        _embed_body,
        grid=(N // R,),
        in_specs=in_specs,
        out_specs=[_rows((R, D))] * 4,
        out_shape=outs,
    )(*in_arrs)


_RK = 128
_CH = 1024
_NCH = N // _CH
_NL = 8
_INF = np.int32(0x7FFFFFFF)
_BIGP = np.int32(1 << 30)


def _knn_body(pos_ref, post_ref, sqr_ref, sqc_ref, pta_ref, sqa_ref,
              out_ref):
    pos_b = pos_ref[...].astype(bf16)
    sqr = sqr_ref[...]
    lane = lax.broadcasted_iota(i32, (_RK, _CH), 1)
    g_local = lane >> 7

    def chunk(c, lists):
        lists = list(lists)
        off = pl.multiple_of(c * _CH, _CH)
        ptc = post_ref[:, pl.ds(off, _CH)].astype(bf16)
        d = sqr + sqc_ref[:, pl.ds(off, _CH)] - 2.0 * lax.dot_general(
            pos_b, ptc, (((1,), (0,)), ((), ())), preferred_element_type=f32)
        b = lax.bitcast_convert_type(d + 0.5, i32)
        keys = lax.bitcast_convert_type(
            (b & jnp.int32(-64)) | (g_local + c * (_CH // 128)), f32)
        for s in range(_CH // 128):
            kg = keys[:, s * 128:(s + 1) * 128]
            for j in range(_NL):
                lo = jnp.minimum(lists[j], kg)
                kg = jnp.maximum(lists[j], kg)
                lists[j] = lo
        return tuple(lists)

    init = tuple(jnp.full((_RK, 128), jnp.inf, f32) for _ in range(_NL))
    lists = lax.fori_loop(0, _NCH, chunk, init)

    da = sqr + sqa_ref[...] - 2.0 * lax.dot_general(
        pos_b, pta_ref[...].astype(bf16), (((1,), (0,)), ((), ())),
        preferred_element_type=f32)
    dmin = jnp.full((_RK, 128), jnp.inf, f32)
    gmin = jnp.zeros((_RK, 128), i32)
    for s in range(M // 128):
        ds_ = da[:, s * 128:(s + 1) * 128]
        cond = ds_ < dmin
        gmin = jnp.where(cond, s, gmin)
        dmin = jnp.minimum(dmin, ds_)
    lane128 = lax.broadcasted_iota(i32, (_RK, 128), 1)
    mu_ = jnp.min(dmin, axis=1, keepdims=True)
    up_col = jnp.min(jnp.where(dmin == mu_, gmin * 128 + lane128, _BIGP),
                     axis=1, keepdims=True)

    cand = jnp.concatenate(lists, axis=1)
    lane_c = lax.broadcasted_iota(i32, (_RK, _NL * 128), 1)
    acc = jnp.zeros((_RK, 24), i32)
    kio = lax.broadcasted_iota(i32, (_RK, 24), 1)
    for kk in range(K):
        m = jnp.min(cand, axis=1, keepdims=True)
        p = jnp.min(jnp.where(cand == m, lane_c, _BIGP), axis=1, keepdims=True)
        col = (lax.bitcast_convert_type(m, i32) & 63) * 128 + (p & 127)
        acc = jnp.where(kio == kk, col, acc)
        cand = jnp.where(lane_c == p, jnp.inf, cand)
    acc = jnp.where(kio == K, up_col, acc)
    out_ref[...] = acc


def _knn(pos8, post8, sqr, sqc, pta, sqa):
    return pl.pallas_call(
        _knn_body,
        grid=(N // _RK,),
        in_specs=[_rows((_RK, 8)), _full((8, N)), _rows((_RK, 1)),
                  _full((1, N)), _full((8, M)), _full((1, M))],
        out_specs=_rows((_RK, 24)),
        out_shape=jax.ShapeDtypeStruct((N, 24), i32),
    )(pos8, post8, sqr, sqc, pta, sqa)


_NW = 32


def _gather_multi(tables, idx):
    B = idx.shape[0]
    T = len(tables)
    bw = B // _NW
    wsum = sum(t.shape[1] for t in tables)
    CH = 128 if wsum <= 384 else 64
    nch = bw // CH
    idx2d = idx.reshape(B // CH, CH)
    mesh = plsc.VectorSubcoreMesh(core_axis_name="c", subcore_axis_name="s")

    scratch = [pltpu.VMEM((nch, CH), i32)]
    for t in tables:
        scratch += [pltpu.VMEM((CH, t.shape[1]), t.dtype)] * 2
    scratch += [pltpu.SemaphoreType.DMA] * (4 * T)

    @functools.partial(
        pl.kernel,
        out_type=[jax.ShapeDtypeStruct((B, t.shape[1]), t.dtype)
                  for t in tables],
        mesh=mesh,
        scratch_types=scratch,
    )
    def gk(*refs):
        tabs = refs[:T]
        idx_hbm = refs[T]
        outs = refs[T + 1:2 * T + 1]
        idx_v = refs[2 * T + 1]
        bufs = refs[2 * T + 2:2 * T + 2 + 2 * T]
        sems = refs[2 * T + 2 + 2 * T:]
        rows = [(bufs[2 * i], bufs[2 * i + 1]) for i in range(T)]
        sg = [(sems[4 * i], sems[4 * i + 1]) for i in range(T)]
        ss = [(sems[4 * i + 2], sems[4 * i + 3]) for i in range(T)]
        wid = lax.axis_index("s") * 2 + lax.axis_index("c")
        base = wid * bw
        pltpu.sync_copy(idx_hbm.at[pl.ds(wid * nch, nch)], idx_v)
        for t in range(T):
            pltpu.async_copy(tabs[t].at[idx_v.at[0]], rows[t][0], sg[t][0])

        def body(c2, carry):
            c = 2 * c2
            for t in range(T):
                pltpu.make_async_copy(tabs[t].at[idx_v.at[c]], rows[t][0],
                                      sg[t][0]).wait()

            @pl.when(c2 > 0)
            def _():
                for t in range(T):
                    pltpu.make_async_copy(rows[t][1],
                                          outs[t].at[pl.ds(base, CH)],
                                          ss[t][1]).wait()

            for t in range(T):
                pltpu.async_copy(tabs[t].at[idx_v.at[c + 1]], rows[t][1],
                                 sg[t][1])
            for t in range(T):
                pltpu.async_copy(rows[t][0],
                                 outs[t].at[pl.ds(base + c * CH, CH)],
                                 ss[t][0])
            for t in range(T):
                pltpu.make_async_copy(tabs[t].at[idx_v.at[c + 1]], rows[t][1],
                                      sg[t][1]).wait()

            @pl.when(c2 < nch // 2 - 1)
            def _():
                for t in range(T):
                    pltpu.make_async_copy(rows[t][0],
                                          outs[t].at[pl.ds(base, CH)],
                                          ss[t][0]).wait()
                    pltpu.async_copy(tabs[t].at[idx_v.at[c + 2]], rows[t][0],
                                     sg[t][0])

            for t in range(T):
                pltpu.async_copy(rows[t][1],
                                 outs[t].at[pl.ds(base + (c + 1) * CH, CH)],
                                 ss[t][1])
            return carry

        lax.fori_loop(0, nch // 2, body, 0)
        for t in range(T):
            pltpu.make_async_copy(rows[t][0], outs[t].at[pl.ds(base, CH)],
                                  ss[t][0]).wait()
            pltpu.make_async_copy(rows[t][1], outs[t].at[pl.ds(base, CH)],
                                  ss[t][1]).wait()

    out = gk(*tables, idx2d)
    return out if isinstance(out, (list, tuple)) else [out]


def _gather_rows(table, idx):
    return _gather_multi([table], idx)[0]


_RA = 128


def _attn_body(q_ref, kg_ref, vg_ref, pg_ref, posp_ref, x_ref,
               rw1, rb1, rw2, rb2, ow, ob, lag, labe, s_ref, st_ref,
               x2_ref):
    RK = _RA * K
    pos_rep = jnp.broadcast_to(posp_ref[...][:, None, :],
                               (_RA, K, 128)).reshape(RK, 128)
    rel = pg_ref[...] - pos_rep
    bias = _mm(jax.nn.gelu(_mm(rel, rw1[...]) + rb1[...]), rw2[...]) + rb2[...]

    q_rep = jnp.broadcast_to(q_ref[...][:, None, :],
                             (_RA, K, D)).reshape(RK, D)
    qb = q_rep.astype(bf16).astype(f32)
    kb = kg_ref[...].astype(bf16).astype(f32)
    prod = qb * kb
    logits = _mm(prod, s_ref[...], prec=lax.Precision.HIGHEST) * SCALE + bias
    l3 = logits.reshape(_RA, K, H)
    mx = jnp.max(l3, axis=1, keepdims=True)
    e = jnp.exp(l3 - mx)
    sm = (e / jnp.sum(e, axis=1, keepdims=True)).reshape(RK, H)
    a_exp = _mm(sm.astype(bf16).astype(f32), st_ref[...],
                prec=lax.Precision.HIGHEST)
    vb = vg_ref[...].astype(bf16).astype(f32)
    o = jnp.sum((a_exp * vb).reshape(_RA, K, D), axis=1)
    out = _mm(o, ow[...]) + ob[...]
    x2_ref[...] = _ln(x_ref[...] + out, lag[...], labe[...])


def _local_attn(qp, kg, vg, pg, posp16, x, w, off):
    smat = jnp.repeat(jnp.eye(H, dtype=f32), DH, axis=0)
    stmat = smat.T
    nrows = kg.shape[0] // K
    ob = off // _RA
    offrows = lambda bshape: pl.BlockSpec(
        bshape, lambda i: (i + ob,) + (0,) * (len(bshape) - 1))
    in_arrs = [qp, kg, vg, pg, posp16, x] + w + [smat, stmat]
    in_specs = ([offrows((_RA, D)), _rows((_RA * K, D)), _rows((_RA * K, D)),
                 _rows((_RA * K, 128)), offrows((_RA, 128)),
                 offrows((_RA, D))] +
                [_full(a.shape) for a in w] +
                [_full((D, H)), _full((H, D))])
    return pl.pallas_call(
        _attn_body,
        grid=(nrows // _RA,),
        in_specs=in_specs,
        out_specs=_rows((_RA, D)),
        out_shape=jax.ShapeDtypeStruct((nrows, D), f32),
    )(*in_arrs)


def _sa_body(xg_ref, pga_ref, posa_ref, swx, swp, sb, sg, sbe, xd_ref):
    RK = _RA * K
    pos_rep = jnp.broadcast_to(posa_ref[...][:, None, :],
                               (_RA, K, 128)).reshape(RK, 128)
    rel = pga_ref[...] - pos_rep
    gin = _mm(xg_ref[...], swx[...]) + _mm(rel, swp[...]) + sb[...]
    g = jax.nn.gelu(_ln(gin, sg[...], sbe[...]))
    xd_ref[...] = jnp.max(g.reshape(_RA, K, D), axis=1)


def _set_abs(xg, pga, posa16, w):
    in_arrs = [xg, pga, posa16] + w
    in_specs = ([_rows((_RA * K, D)), _rows((_RA * K, 128)),
                 _rows((_RA, 128))] + [_full(a.shape) for a in w])
    return pl.pallas_call(
        _sa_body,
        grid=(M // _RA,),
        in_specs=in_specs,
        out_specs=_rows((_RA, D)),
        out_shape=jax.ShapeDtypeStruct((M, D), f32),
    )(*in_arrs)


def _ga_attn_body(xd_ref, qw, qb, kw, kb, vw, vb, og_ref):
    qh = (_mm(xd_ref[...], qw[0]) + qb[0]).astype(bf16)
    kh = (_mm(xd_ref[...], kw[0]) + kb[0]).astype(bf16)
    vh = _mm(xd_ref[...], vw[0]) + vb[0]
    s = lax.dot_general(qh, kh, (((1,), (1,)), ((), ())),
                        preferred_element_type=f32) * SCALE
    mx = jnp.max(s, axis=1, keepdims=True)
    e = jnp.exp(s - mx)
    a = e / jnp.sum(e, axis=1, keepdims=True)
    og_ref[0] = _mm(a, vh)


def _ga_post_body(xd_ref, og_ref, gow, gob, n1g, n1be, f1w, f1b, f2w, f2b,
                  n2g, n2be, xd2_ref):
    og = gob[...]
    for h in range(H):
        og = og + _mm(og_ref[h], gow[h])
    xd1 = _ln(xd_ref[...] + og, n1g[...], n1be[...])
    ff = _mm(jax.nn.gelu(_mm(xd1, f1w[...]) + f1b[...]), f2w[...]) + f2b[...]
    xd2_ref[...] = _ln(xd1 + ff, n2g[...], n2be[...])


def _global_attn(xd, qkv_w, qkv_b, w_post):
    qkvw3 = qkv_w.reshape(D, 3, H, DH).transpose(1, 2, 0, 3).reshape(
        3 * H, D, DH)
    qkvb3 = qkv_b.reshape(3, H, 1, DH).reshape(3 * H, 1, DH)
    wspec = pl.BlockSpec((1, D, DH), lambda h: (h, 0, 0))
    bspec = pl.BlockSpec((1, 1, DH), lambda h: (h, 0, 0))
    og3 = pl.pallas_call(
        _ga_attn_body,
        grid=(H,),
        in_specs=[_full((M, D)),
                  pl.BlockSpec((1, D, DH), lambda h: (h, 0, 0)),
                  pl.BlockSpec((1, 1, DH), lambda h: (h, 0, 0)),
                  pl.BlockSpec((1, D, DH), lambda h: (H + h, 0, 0)),
                  pl.BlockSpec((1, 1, DH), lambda h: (H + h, 0, 0)),
                  pl.BlockSpec((1, D, DH), lambda h: (2 * H + h, 0, 0)),
                  pl.BlockSpec((1, 1, DH), lambda h: (2 * H + h, 0, 0))],
        out_specs=pl.BlockSpec((1, M, DH), lambda h: (h, 0, 0)),
        out_shape=jax.ShapeDtypeStruct((H, M, DH), f32),
    )(xd, qkvw3, qkvb3, qkvw3, qkvb3, qkvw3, qkvb3)
    gow3 = w_post[0].reshape(H, DH, D)
    return pl.pallas_call(
        _ga_post_body,
        grid=(1,),
        in_specs=[_full((M, D)), _full((H, M, DH)), _full((H, DH, D))] +
                 [_full(a.shape) for a in w_post[1:]],
        out_specs=_full((M, D)),
        out_shape=jax.ShapeDtypeStruct((M, D), f32),
    )(xd, og3, gow3, *w_post[1:])


def _fp_body(xdg_ref, x2_ref, w1a, w1b, b1, g1, be1, w2, b2, g2, be2, y_ref):
    cat = _mm(xdg_ref[...], w1a[...]) + _mm(x2_ref[...], w1b[...]) + b1[...]
    y = jax.nn.gelu(_ln(cat, g1[...], be1[...]))
    y = jax.nn.gelu(_ln(_mm(y, w2[...]) + b2[...], g2[...], be2[...]))
    y_ref[...] = y


def _fprop(xdg, x2, w):
    R = 512
    in_arrs = [xdg, x2] + w
    in_specs = ([_rows((R, D)), _rows((R, D))] + [_full(a.shape) for a in w])
    return pl.pallas_call(
        _fp_body,
        grid=(N // R,),
        in_specs=in_specs,
        out_specs=_rows((R, D)),
        out_shape=jax.ShapeDtypeStruct((N, D), f32),
    )(*in_arrs)


def kernel(pos, feat, params):
    p = params
    row = lambda a: a.reshape(1, -1)

    posp128 = jnp.pad(pos, ((0, 0), (0, 125)))
    pos8 = posp128[:, :8]
    post8 = pos8.T
    sq = jnp.sum(pos * pos, -1)
    sqr = sq.reshape(N, 1)
    sqc = sq.reshape(1, N)

    emb_w = [p['ce_w1'], row(p['ce_b1']), row(p['ce_g']), row(p['ce_be']),
             p['ce_w2'], row(p['ce_b2']),
             p['fe_w1'], row(p['fe_b1']), row(p['fe_g']), row(p['fe_be']),
             p['fe_w2'], row(p['fe_b2']),
             p['fu_w'][:D], p['fu_w'][D:], row(p['fu_b']), row(p['fu_g']),
             row(p['fu_be']),
             p['q_w'], row(p['q_b']), p['k_w'], row(p['k_b']),
             p['v_w'], row(p['v_b'])]
    x, qp, kp, vp = _embed(pos, feat, emb_w)

    knn = _knn(pos8, post8, sqr, sqc, post8[:, ::4], sqc[:, ::4])
    idx = knn[:, :K]
    up = knn[:, K]

    idxf = idx.reshape(N * K)
    rp_w1p = jnp.pad(p['rp_w1'], ((0, 125), (0, 0)))
    attn_w = [rp_w1p, row(p['rp_b1']), p['rp_w2'], row(p['rp_b2']),
              p['o_w'], row(p['o_b']), row(p['la_g']), row(p['la_be'])]
    P = 4
    step = N * K // P
    parts = []
    for j in range(P):
        kgj, vgj, pgj = _gather_multi([kp, vp, posp128],
                                      idxf[j * step:(j + 1) * step])
        parts.append(_local_attn(qp, kgj, vgj, pgj, posp128, x, attn_w,
                                 j * (N // P)))
    x2 = jnp.concatenate(parts, axis=0)

    gi = idx[::4].reshape(M * K)
    xg, pga = _gather_multi([x2, posp128], gi)
    posa16 = posp128[::4]
    saw_p = jnp.pad(p['sa_w'][D:], ((0, 125), (0, 0)))
    sa_w = [p['sa_w'][:D], saw_p, row(p['sa_b']), row(p['sa_g']),
            row(p['sa_be'])]
    xd = _set_abs(xg, pga, posa16, sa_w)

    ga_post = [p['go_w'], row(p['go_b']), row(p['n1_g']), row(p['n1_be']),
               p['f1_w'], row(p['f1_b']), p['f2_w'], row(p['f2_b']),
               row(p['n2_g']), row(p['n2_be'])]
    xd2 = _global_attn(xd, p['qkv_w'], p['qkv_b'], ga_post)

    xdg = _gather_rows(xd2, up)

    fp_w = [p['fp_w1'][:D], p['fp_w1'][D:], row(p['fp_b1']), row(p['fp_g1']),
            row(p['fp_be1']), p['fp_w2'], row(p['fp_b2']), row(p['fp_g2']),
            row(p['fp_be2'])]
    return _fprop(xdg, x2, fp_w)
